# baseline (speedup 1.0000x reference)
import jax
import jax.numpy as jnp
from jax import lax
from jax.experimental import pallas as pl
from jax.experimental.pallas import tpu as pltpu
from jax.experimental.pallas import tpu_sc as plsc

B = 16384
D = 16
NC = 2
NS = 16
L = 16
NW = NC * NS
BPW = B // NW
NBLK = BPW // L


def _mf_body(uid_table, iid_table, x_hbm, out_hbm,
             x_v, uidx_v, iidx_v, u_rows, i_rows, out_v, sem):
    wid = lax.axis_index("s") * NC + lax.axis_index("c")
    base = wid * BPW
    pltpu.sync_copy(x_hbm.at[pl.ds(base, BPW), :], x_v)

    lanes = lax.iota(jnp.int32, L)
    zeros = jnp.zeros((L,), jnp.int32)
    ones = jnp.ones((L,), jnp.int32)

    def deint_body(k, carry):
        off = k * L
        rows = off + lanes
        uidx_v[pl.ds(off, L)] = plsc.load_gather(x_v, [rows, zeros])
        iidx_v[pl.ds(off, L)] = plsc.load_gather(x_v, [rows, ones])
        return carry

    lax.fori_loop(0, BPW // L, deint_body, 0)

    cu = pltpu.async_copy(uid_table.at[uidx_v], u_rows, sem)
    ci = pltpu.async_copy(iid_table.at[iidx_v], i_rows, sem)
    cu.wait()
    ci.wait()

    def blk_body(blk, carry):
        row0 = blk * L
        rows = row0 + lanes
        acc = jnp.zeros((L,), jnp.float32)
        for d in range(D):
            cols = jnp.full((L,), d, jnp.int32)
            u = plsc.load_gather(u_rows, [rows, cols])
            v = plsc.load_gather(i_rows, [rows, cols])
            acc = acc + u * v
        out_v[pl.ds(row0, L)] = acc
        return carry

    lax.fori_loop(0, NBLK, blk_body, 0)
    pltpu.sync_copy(out_v, out_hbm.at[pl.ds(base, BPW)])


@jax.jit
def kernel(x, uid_table, iid_table):
    k = pl.kernel(
        _mf_body,
        out_type=jax.ShapeDtypeStruct((B,), jnp.float32),
        mesh=plsc.VectorSubcoreMesh(core_axis_name="c", subcore_axis_name="s"),
        scratch_types=[
            pltpu.VMEM((BPW, 2), jnp.int32),
            pltpu.VMEM((BPW,), jnp.int32),
            pltpu.VMEM((BPW,), jnp.int32),
            pltpu.VMEM((BPW, D), jnp.float32),
            pltpu.VMEM((BPW, D), jnp.float32),
            pltpu.VMEM((BPW,), jnp.float32),
            pltpu.SemaphoreType.DMA,
        ],
        compiler_params=pltpu.CompilerParams(
            use_tc_tiling_on_sc=False, needs_layout_passes=False
        ),
    )
    return k(uid_table, iid_table, x)


# speedup vs baseline: 6.1909x; 6.1909x over previous
import jax
import jax.numpy as jnp
from jax import lax
from jax.experimental import pallas as pl
from jax.experimental.pallas import tpu as pltpu
from jax.experimental.pallas import tpu_sc as plsc

B = 16384
D = 16
NC = 2
NS = 16
L = 16
NW = NC * NS
BPW = B // NW
NCH = BPW // L


def _mf_body(ut_hbm, it_hbm, xt_hbm, out_hbm,
             uidx_v, iidx_v, u_st, i_st, out_v, sem):
    wid = lax.axis_index("s") * NC + lax.axis_index("c")
    base = wid * BPW
    pltpu.sync_copy(xt_hbm.at[0, pl.ds(base, BPW)], uidx_v)
    pltpu.sync_copy(xt_hbm.at[1, pl.ds(base, BPW)], iidx_v)

    lanes = lax.iota(jnp.int32, L)

    def chunk(ch, carry):
        j0 = ch * L
        rv = uidx_v[pl.ds(j0, L)]
        qv = iidx_v[pl.ds(j0, L)]

        copies = []
        for jj in range(L):
            r = pl.multiple_of((rv[jj] >> 7) << 7, 128)
            q = pl.multiple_of((qv[jj] >> 7) << 7, 128)
            copies.append(pltpu.async_copy(
                ut_hbm.at[:, :, pl.ds(r, 128)], u_st.at[jj], sem))
            copies.append(pltpu.async_copy(
                it_hbm.at[:, :, pl.ds(q, 128)], i_st.at[jj], sem))
        for c in copies:
            c.wait()

        ucols = rv & 127
        icols = qv & 127
        acc = jnp.zeros((L,), jnp.float32)
        for b in range(2):
            for s in range(8):
                bb = jnp.full((L,), b, jnp.int32)
                ss = jnp.full((L,), s, jnp.int32)
                u = plsc.load_gather(u_st, [lanes, bb, ss, ucols])
                v = plsc.load_gather(i_st, [lanes, bb, ss, icols])
                acc = acc + u * v
        out_v[pl.ds(j0, L)] = acc
        return carry

    lax.fori_loop(0, NCH, chunk, 0)
    pltpu.sync_copy(out_v, out_hbm.at[pl.ds(base, BPW)])


@jax.jit
def kernel(x, uid_table, iid_table):
    ut = uid_table.T.reshape(2, 8, uid_table.shape[0])
    it = iid_table.T.reshape(2, 8, iid_table.shape[0])
    k = pl.kernel(
        _mf_body,
        out_type=jax.ShapeDtypeStruct((B,), jnp.float32),
        mesh=plsc.VectorSubcoreMesh(core_axis_name="c", subcore_axis_name="s"),
        scratch_types=[
            pltpu.VMEM((BPW,), jnp.int32),
            pltpu.VMEM((BPW,), jnp.int32),
            pltpu.VMEM((L, 2, 8, 128), jnp.float32),
            pltpu.VMEM((L, 2, 8, 128), jnp.float32),
            pltpu.VMEM((BPW,), jnp.float32),
            pltpu.SemaphoreType.DMA,
        ],
        compiler_params=pltpu.CompilerParams(needs_layout_passes=False),
    )
    return k(ut, it, x.T)
